# baseline (speedup 1.0000x reference)
import functools

import jax
import jax.numpy as jnp
from jax import lax
from jax.experimental import pallas as pl
from jax.experimental.pallas import tpu as pltpu
from jax.experimental.pallas import tpu_sc as plsc

N_ROWS = 1_000_000
D = 64
B_TOTAL = 4096 * 200

NC = 2
NS = 16
NW = NC * NS
B_PER_W = B_TOTAL // NW
CHUNK = 800
N_CHUNKS = B_PER_W // CHUNK


def _embed_body(table_hbm, idx_hbm, out_hbm, idx_v, rows_v, sem):
    wid = lax.axis_index("s") * NC + lax.axis_index("c")
    base = wid * B_PER_W

    def step(i, carry):
        off = base + i * CHUNK
        pltpu.sync_copy(idx_hbm.at[pl.ds(off, CHUNK)], idx_v)
        pltpu.async_copy(table_hbm.at[idx_v], rows_v, sem).wait()
        pltpu.sync_copy(rows_v, out_hbm.at[pl.ds(off, CHUNK)])
        return carry

    lax.fori_loop(0, N_CHUNKS, step, 0)


@jax.jit
def _embed(x_flat, table):
    mesh = plsc.VectorSubcoreMesh(core_axis_name="c", subcore_axis_name="s")
    out = pl.kernel(
        _embed_body,
        out_type=jax.ShapeDtypeStruct((B_TOTAL, D), jnp.float32),
        mesh=mesh,
        scratch_types=[
            pltpu.VMEM((CHUNK,), jnp.int32),
            pltpu.VMEM((CHUNK, D), jnp.float32),
            pltpu.SemaphoreType.DMA,
        ],
        compiler_params=pltpu.CompilerParams(use_tc_tiling_on_sc=False),
    )(table, x_flat)
    return out


def kernel(x, table):
    out = _embed(x.reshape(-1), table)
    return out.reshape(x.shape[0], x.shape[1], D)


# speedup vs baseline: 1.0256x; 1.0256x over previous
import functools

import jax
import jax.numpy as jnp
from jax import lax
from jax.experimental import pallas as pl
from jax.experimental.pallas import tpu as pltpu
from jax.experimental.pallas import tpu_sc as plsc

N_ROWS = 1_000_000
D = 64
B_TOTAL = 4096 * 200

NC = 2
NS = 16
NW = NC * NS
B_PER_W = B_TOTAL // NW
CHUNK = 800
N_CHUNKS = B_PER_W // CHUNK


def _embed_body(table_hbm, idx_hbm, out_hbm,
                idx_all, buf0, buf1, gsem0, gsem1, ssem0, ssem1):
    wid = lax.axis_index("s") * NC + lax.axis_index("c")
    base = wid * B_PER_W
    pltpu.sync_copy(idx_hbm.at[pl.ds(base, B_PER_W)], idx_all)

    bufs = (buf0, buf1)
    gsems = (gsem0, gsem1)
    ssems = (ssem0, ssem1)

    def gather(i, b):
        pltpu.async_copy(
            table_hbm.at[idx_all.at[pl.ds(i * CHUNK, CHUNK)]], bufs[b], gsems[b])

    def store(i, b):
        pltpu.async_copy(bufs[b], out_hbm.at[pl.ds(base + i * CHUNK, CHUNK)],
                         ssems[b])

    def wait_g(b):
        pltpu.make_async_copy(out_hbm.at[pl.ds(base, CHUNK)], bufs[b],
                              gsems[b]).wait()

    def wait_s(b):
        pltpu.make_async_copy(bufs[b], out_hbm.at[pl.ds(base, CHUNK)],
                              ssems[b]).wait()

    gather(0, 0)
    gather(1, 1)
    wait_g(0)
    store(0, 0)

    @pl.loop(0, (N_CHUNKS - 2) // 2)
    def _(k):
        i1 = 2 * k + 1
        wait_g(1)
        store(i1, 1)
        wait_s(0)
        gather(i1 + 1, 0)
        i2 = 2 * k + 2
        wait_g(0)
        store(i2, 0)
        wait_s(1)
        gather(i2 + 1, 1)

    wait_g(1)
    store(N_CHUNKS - 1, 1)
    wait_s(0)
    wait_s(1)


@jax.jit
def _embed(x_flat, table):
    mesh = plsc.VectorSubcoreMesh(core_axis_name="c", subcore_axis_name="s")
    out = pl.kernel(
        _embed_body,
        out_type=jax.ShapeDtypeStruct((B_TOTAL, D), jnp.float32),
        mesh=mesh,
        scratch_types=[
            pltpu.VMEM((B_PER_W,), jnp.int32),
            pltpu.VMEM((CHUNK, D), jnp.float32),
            pltpu.VMEM((CHUNK, D), jnp.float32),
            pltpu.SemaphoreType.DMA,
            pltpu.SemaphoreType.DMA,
            pltpu.SemaphoreType.DMA,
            pltpu.SemaphoreType.DMA,
        ],
        compiler_params=pltpu.CompilerParams(use_tc_tiling_on_sc=False),
    )(table, x_flat)
    return out


def kernel(x, table):
    out = _embed(x.reshape(-1), table)
    return out.reshape(x.shape[0], x.shape[1], D)


# speedup vs baseline: 1.4661x; 1.4296x over previous
import functools

import jax
import jax.numpy as jnp
from jax import lax
from jax.experimental import pallas as pl
from jax.experimental.pallas import tpu as pltpu
from jax.experimental.pallas import tpu_sc as plsc

N_ROWS = 1_000_000
D = 64
B_TOTAL = 4096 * 200

NC = 2
NS = 16
NW = NC * NS
B_PER_W = B_TOTAL // NW
CHUNK = 800
N_CHUNKS = B_PER_W // CHUNK


def _embed_body(table_hbm, idx_hbm, out_hbm,
                idx_all, buf0, buf1, gsem0, gsem1, ssem0, ssem1):
    wid = lax.axis_index("s") * NC + lax.axis_index("c")
    base = wid * B_PER_W
    pltpu.sync_copy(idx_hbm.at[pl.ds(base, B_PER_W)], idx_all)

    bufs = (buf0, buf1)
    gsems = (gsem0, gsem1)
    ssems = (ssem0, ssem1)

    def gather(i, b):
        pltpu.async_copy(
            table_hbm.at[idx_all.at[pl.ds(i * CHUNK, CHUNK)]], bufs[b], gsems[b])

    def store(i, b):
        pltpu.async_copy(
            bufs[b],
            out_hbm.at[pl.ds(base + i * CHUNK, CHUNK), pl.ds(0, D)],
            ssems[b])

    def wait_g(b):
        pltpu.make_async_copy(table_hbm.at[pl.ds(base, CHUNK)], bufs[b],
                              gsems[b]).wait()

    def wait_s(b):
        pltpu.make_async_copy(bufs[b],
                              out_hbm.at[pl.ds(base, CHUNK), pl.ds(0, D)],
                              ssems[b]).wait()

    gather(0, 0)
    gather(1, 1)
    wait_g(0)
    store(0, 0)

    @pl.loop(0, (N_CHUNKS - 2) // 2)
    def _(k):
        i1 = 2 * k + 1
        wait_g(1)
        store(i1, 1)
        wait_s(0)
        gather(i1 + 1, 0)
        i2 = 2 * k + 2
        wait_g(0)
        store(i2, 0)
        wait_s(1)
        gather(i2 + 1, 1)

    wait_g(1)
    store(N_CHUNKS - 1, 1)
    wait_s(0)
    wait_s(1)


@jax.jit
def _embed(x2_flat, table2):
    mesh = plsc.VectorSubcoreMesh(core_axis_name="c", subcore_axis_name="s")
    out = pl.kernel(
        _embed_body,
        out_type=jax.ShapeDtypeStruct((B_TOTAL, 2 * D), jnp.float32),
        mesh=mesh,
        scratch_types=[
            pltpu.VMEM((B_PER_W,), jnp.int32),
            pltpu.VMEM((CHUNK, D), jnp.float32),
            pltpu.VMEM((CHUNK, D), jnp.float32),
            pltpu.SemaphoreType.DMA,
            pltpu.SemaphoreType.DMA,
            pltpu.SemaphoreType.DMA,
            pltpu.SemaphoreType.DMA,
        ],
        compiler_params=pltpu.CompilerParams(use_tc_tiling_on_sc=False),
    )(table2, x2_flat)
    return out


def kernel(x, table):
    table2 = jnp.pad(table, ((0, 0), (0, D))).reshape(2 * N_ROWS, D)
    x2 = x.reshape(-1) * 2
    out = _embed(x2, table2)
    return out.reshape(x.shape[0], x.shape[1], 2 * D)[:, :, :D]
